# Initial kernel scaffold; baseline (speedup 1.0000x reference)
import jax
import jax.numpy as jnp
from jax.experimental import pallas as pl


def kernel(x, features_of_edges, node_indexes_connected_by_edges, batch, params):
    raise NotImplementedError("write your pallas kernel here")


# speedup vs baseline: 1.8124x; 1.8124x over previous
import functools

import jax
import jax.numpy as jnp
from jax import lax
from jax.experimental import pallas as pl
from jax.experimental.pallas import tpu as pltpu
from jax.experimental.pallas import tpu_sc as plsc

N = 100000
NP = 100352
E = 1600000
EP = 1638400
NC, NS = 2, 16
BE = 2048
GE = (E + BE - 1) // BE
OFF_DST = EP // BE
BN = 2048
GN = NP // BN


def _mesh():
    return plsc.VectorSubcoreMesh(core_axis_name="c", subcore_axis_name="s",
                                  num_cores=NC, num_subcores=NS)


def _relu(v):
    return jnp.maximum(v, 0.0)


def _dot(a, b):
    return jnp.dot(a, b, preferred_element_type=jnp.float32)


def _node_enc_body(x_ref, w1, b1, w2, b2, w3, b3, w4, b4, out_ref):
    h = _relu(_dot(x_ref[...], w1[...]) + b1[...])
    h = _relu(_dot(h, w2[...]) + b2[...])
    h = _relu(_dot(h, w3[...]) + b3[...])
    enc = _dot(h, w4[...]) + b4[...]
    out_ref[...] = jnp.pad(enc, ((0, 0), (0, 108)))


def _node_encode(x, ws, bs):
    full = lambda shape: pl.BlockSpec(shape, lambda i: (0,) * len(shape))
    in_specs = [pl.BlockSpec((BN, 4), lambda i: (i, 0))]
    args = [x]
    for w, b in zip(ws, bs):
        in_specs += [full(w.shape), full(b.shape)]
        args += [w, b]
    return pl.pallas_call(
        _node_enc_body,
        grid=(GN,),
        in_specs=in_specs,
        out_specs=pl.BlockSpec((BN, 128), lambda i: (i, 0)),
        out_shape=jax.ShapeDtypeStruct((NP, 128), jnp.float32),
    )(*args)


def _sc_gather_body(table, idx2d, out, idx_v, rows_v, cmp_v, sem):
    c = lax.axis_index("c")
    s = lax.axis_index("s")
    wid = s * NC + c
    base = wid * 800

    def body(j, carry):
        r0 = base + j * 2
        pltpu.sync_copy(idx2d.at[pl.ds(r0, 2)], idx_v)
        waits = []
        for r in range(2):
            waits.append(
                pltpu.async_copy(table.at[idx_v.at[r]],
                                 rows_v.at[pl.ds(r * 128, 128)], sem))
        for wdesc in waits:
            wdesc.wait()
        for e in range(256):
            cmp_v[e, pl.ds(0, 16)] = rows_v[e, pl.ds(0, 16)]
            cmp_v[e, pl.ds(16, 16)] = rows_v[e, pl.ds(16, 16)]
        pltpu.sync_copy(cmp_v, out.at[pl.ds(r0 * 128, 256)])
        return carry

    lax.fori_loop(0, 400, body, 0)


def _sc_gather(table128, idx2d):
    kern = pl.kernel(
        _sc_gather_body,
        out_type=jax.ShapeDtypeStruct((2 * EP, 32), jnp.float32),
        mesh=_mesh(),
        scratch_types=[
            pltpu.VMEM((2, 128), jnp.int32),
            pltpu.VMEM((256, 128), jnp.float32),
            pltpu.VMEM((256, 32), jnp.float32),
            pltpu.SemaphoreType.DMA,
        ],
    )
    return kern(table128, idx2d)


def _edge_body(f_ref, gs_ref, gd_ref, w1, b1, w2, b2, w3, b3, w4, b4,
               wm1, bm1, wm2, bm2, lo_ref, hi_ref):
    h = _relu(_dot(f_ref[...], w1[...]) + b1[...])
    h = _relu(_dot(h, w2[...]) + b2[...])
    h = _relu(_dot(h, w3[...]) + b3[...])
    ence = _dot(h, w4[...]) + b4[...]
    cat = jnp.concatenate([gs_ref[:, :20], gd_ref[:, :20], ence], axis=1)
    hm = _relu(_dot(cat, wm1[...]) + bm1[...])
    msg = _dot(hm, wm2[...]) + bm2[...]
    z = jnp.zeros((msg.shape[0], 6), jnp.float32)
    lo_ref[...] = jnp.concatenate([msg[:, :10], z], axis=1)
    hi_ref[...] = jnp.concatenate([msg[:, 10:], z], axis=1)


def _edge_stage(feats, g, we, be_, wm1, bm1, wm2, bm2):
    full = lambda shape: pl.BlockSpec(shape, lambda i: (0,) * len(shape))
    in_specs = [
        pl.BlockSpec((BE, 5), lambda i: (i, 0)),
        pl.BlockSpec((BE, 32), lambda i: (i, 0)),
        pl.BlockSpec((BE, 32), lambda i: (i + OFF_DST, 0)),
    ]
    args = [feats, g, g]
    for w, b in zip(we, be_):
        in_specs += [full(w.shape), full(b.shape)]
        args += [w, b]
    for t in (wm1, bm1, wm2, bm2):
        in_specs.append(full(t.shape))
        args.append(t)
    return pl.pallas_call(
        _edge_body,
        grid=(GE,),
        in_specs=in_specs,
        out_specs=[pl.BlockSpec((BE, 16), lambda i: (i, 0)),
                   pl.BlockSpec((BE, 16), lambda i: (i, 0))],
        out_shape=[jax.ShapeDtypeStruct((EP, 16), jnp.float32),
                   jax.ShapeDtypeStruct((EP, 16), jnp.float32)],
    )(*args)


def _sc_scatter(mlo_pk, mhi_pk, dst2d):
    HR = NP // 2
    DTOP = HR + 128

    def body(mlo_hbm, mhi_hbm, dst2d_hbm, plo_hbm, phi_hbm,
             idx_v, idx2_v, raw_v, msg_v, zv_v, av_v, pk_v, acc, sem):
        c = lax.axis_index("c")
        s = lax.axis_index("s")
        wid = s * NC + c
        stripe = HR // NS
        r0 = s * stripe
        base = wid * 400

        def zfill(i, carry):
            zv_v[i, pl.ds(0, 16)] = jnp.zeros((16,), jnp.float32)
            return carry
        lax.fori_loop(0, 784, zfill, 0)

        for half in (0, 1):
            off = half * HR
            for msg_hbm, part_hbm in ((mlo_hbm, plo_hbm), (mhi_hbm, phi_hbm)):
                def zloop(k, carry):
                    pltpu.sync_copy(zv_v, acc.at[pl.ds(r0 + k * 784, 784)])
                    return carry
                lax.fori_loop(0, 4, zloop, 0)
                pltpu.sync_copy(zv_v.at[pl.ds(0, 8)],
                                acc.at[pl.ds(HR + s * 8, 8)])
                plsc.subcore_barrier()

                def loop(j, carry):
                    rr = base + j
                    pltpu.sync_copy(dst2d_hbm.at[pl.ds(rr, 1)], idx_v)
                    pltpu.sync_copy(msg_hbm.at[pl.ds(rr * 16, 16)], raw_v)
                    for t in range(8):
                        v = idx_v[0, pl.ds(t * 16, 16)]
                        lo = v - off
                        oob = (lo < 0) | (lo >= HR)
                        dump = HR + (v & 127)
                        idx2_v[0, pl.ds(t * 16, 16)] = jnp.where(oob, dump, lo)
                    for r in range(128):
                        msg_v[r, pl.ds(0, 16)] = raw_v[r // 8,
                                                       pl.ds((r % 8) * 16, 16)]
                    pltpu.sync_copy(msg_v, acc.at[idx2_v.at[0]], add=True)
                    return carry

                lax.fori_loop(0, 400, loop, 0)
                plsc.subcore_barrier()

                def wloop(k, carry):
                    pltpu.sync_copy(acc.at[pl.ds(r0 + k * 784, 784)], av_v)
                    for r in range(98):
                        for q in range(8):
                            pk_v[r, pl.ds(q * 16, 16)] = av_v[r * 8 + q,
                                                              pl.ds(0, 16)]
                    pltpu.sync_copy(
                        pk_v,
                        part_hbm.at[c].at[pl.ds(half * 6272 + s * 392 + k * 98,
                                                98)])
                    return carry
                lax.fori_loop(0, 4, wloop, 0)
                plsc.subcore_barrier()

    kern = pl.kernel(
        body,
        out_type=[jax.ShapeDtypeStruct((NC, NP // 8, 128), jnp.float32),
                  jax.ShapeDtypeStruct((NC, NP // 8, 128), jnp.float32)],
        mesh=_mesh(),
        scratch_types=[
            pltpu.VMEM((1, 128), jnp.int32),
            pltpu.VMEM((1, 128), jnp.int32),
            pltpu.VMEM((16, 128), jnp.float32),
            pltpu.VMEM((128, 16), jnp.float32),
            pltpu.VMEM((784, 16), jnp.float32),
            pltpu.VMEM((784, 16), jnp.float32),
            pltpu.VMEM((98, 128), jnp.float32),
            pltpu.VMEM_SHARED((NP // 2 + 128, 16), jnp.float32),
            pltpu.SemaphoreType.DMA,
        ],
        compiler_params=pltpu.CompilerParams(use_tc_tiling_on_sc=False),
    )
    return kern(mlo_pk, mhi_pk, dst2d)


def _node_out_body(enc_ref, lo0_ref, lo1_ref, hi0_ref, hi1_ref, v1, c1, v2, c2,
                   d1, e1, d2, e2, d3, e3, d4, e4, out_ref):
    enc = enc_ref[:, :20]
    agg_lo = (lo0_ref[0] + lo1_ref[0])[:, :10]
    agg_hi = (hi0_ref[0] + hi1_ref[0])[:, :10]
    cat = jnp.concatenate([enc, agg_lo, agg_hi], axis=1)
    h = _relu(_dot(cat, v1[...]) + c1[...])
    pred = enc + _dot(h, v2[...]) + c2[...]
    h = _relu(_dot(pred, d1[...]) + e1[...])
    h = _relu(_dot(h, d2[...]) + e2[...])
    h = _relu(_dot(h, d3[...]) + e3[...])
    out_ref[...] = _dot(h, d4[...]) + e4[...]


def _node_out(table128, parts_lo, parts_hi, v1, c1, v2, c2, wd, bd):
    full = lambda shape: pl.BlockSpec(shape, lambda i: (0,) * len(shape))
    in_specs = [
        pl.BlockSpec((BN, 128), lambda i: (i, 0)),
        pl.BlockSpec((1, BN, 16), lambda i: (0, i, 0)),
        pl.BlockSpec((1, BN, 16), lambda i: (1, i, 0)),
        pl.BlockSpec((1, BN, 16), lambda i: (0, i, 0)),
        pl.BlockSpec((1, BN, 16), lambda i: (1, i, 0)),
    ]
    args = [table128, parts_lo, parts_lo, parts_hi, parts_hi]
    for t in (v1, c1, v2, c2):
        in_specs.append(full(t.shape))
        args.append(t)
    for w, b in zip(wd, bd):
        in_specs += [full(w.shape), full(b.shape)]
        args += [w, b]
    return pl.pallas_call(
        _node_out_body,
        grid=(GN,),
        in_specs=in_specs,
        out_specs=pl.BlockSpec((BN, 3), lambda i: (i, 0)),
        out_shape=jax.ShapeDtypeStruct((N, 3), jnp.float32),
    )(*args)


def kernel(x, features_of_edges, node_indexes_connected_by_edges, batch, params):
    p = params
    r1 = lambda b: b.reshape(1, -1)

    w1m = p["mp_edge_model"]["Ws"][0]
    b1m = p["mp_edge_model"]["bs"][0]
    w2m = p["mp_edge_model"]["Ws"][1]
    b2m = p["mp_edge_model"]["bs"][1]

    we = p["edge_encoder"]["Ws"]
    bbe = p["edge_encoder"]["bs"]

    src = node_indexes_connected_by_edges[0]
    dst = node_indexes_connected_by_edges[1]
    pad_i = jnp.arange(EP - E, dtype=jnp.int32)
    gpad = pad_i % N
    gidx = jnp.concatenate([src, gpad, dst, gpad])
    idx2d = gidx.reshape(25600, 128)
    dst_pad = jnp.concatenate([dst, N + pad_i % (NP - N)])
    dst2d = dst_pad.reshape(12800, 128)

    table128 = _node_encode(x, p["node_encoder"]["Ws"],
                            [r1(b) for b in p["node_encoder"]["bs"]])
    g = _sc_gather(table128, idx2d)
    mlo, mhi = _edge_stage(features_of_edges, g,
                           we, [r1(b) for b in bbe],
                           w1m, r1(b1m), w2m, r1(b2m))
    mlo_pk = mlo.reshape(EP // 8, 128)
    mhi_pk = mhi.reshape(EP // 8, 128)
    plo_pk, phi_pk = _sc_scatter(mlo_pk, mhi_pk, dst2d)
    parts_lo = plo_pk.reshape(NC, NP, 16)
    parts_hi = phi_pk.reshape(NC, NP, 16)

    v1 = p["mp_node_model"]["Ws"][0]
    c1 = p["mp_node_model"]["bs"][0]
    v2 = p["mp_node_model"]["Ws"][1]
    c2 = p["mp_node_model"]["bs"][1]
    out = _node_out(table128, parts_lo, parts_hi, v1, r1(c1), v2, r1(c2),
                    p["node_decoder"]["Ws"],
                    [r1(b) for b in p["node_decoder"]["bs"]])
    return out


# speedup vs baseline: 2.3135x; 1.2765x over previous
import functools

import jax
import jax.numpy as jnp
from jax import lax
from jax.experimental import pallas as pl
from jax.experimental.pallas import tpu as pltpu
from jax.experimental.pallas import tpu_sc as plsc

N = 100000
NP = 100352
E = 1600000
EP = 1638400
NC, NS = 2, 16
BE = 2048
GE = (E + BE - 1) // BE
OFF_DST = EP // BE
BN = 2048
GN = NP // BN


def _mesh():
    return plsc.VectorSubcoreMesh(core_axis_name="c", subcore_axis_name="s",
                                  num_cores=NC, num_subcores=NS)


def _relu(v):
    return jnp.maximum(v, 0.0)


def _dot(a, b):
    return jnp.dot(a, b, preferred_element_type=jnp.float32)


def _node_enc_body(x_ref, w1, b1, w2, b2, w3, b3, w4, b4, out_ref):
    h = _relu(_dot(x_ref[...], w1[...]) + b1[...])
    h = _relu(_dot(h, w2[...]) + b2[...])
    h = _relu(_dot(h, w3[...]) + b3[...])
    enc = _dot(h, w4[...]) + b4[...]
    out_ref[...] = jnp.pad(enc, ((0, 0), (0, 108)))


def _node_encode(x, ws, bs):
    full = lambda shape: pl.BlockSpec(shape, lambda i: (0,) * len(shape))
    in_specs = [pl.BlockSpec((BN, 4), lambda i: (i, 0))]
    args = [x]
    for w, b in zip(ws, bs):
        in_specs += [full(w.shape), full(b.shape)]
        args += [w, b]
    return pl.pallas_call(
        _node_enc_body,
        grid=(GN,),
        in_specs=in_specs,
        out_specs=pl.BlockSpec((BN, 128), lambda i: (i, 0)),
        out_shape=jax.ShapeDtypeStruct((NP, 128), jnp.float32),
    )(*args)


def _sc_gather_body(table, idx2d, out, idx_v, rows_a, rows_b, cmp_v,
                    sem_a, sem_b):
    c = lax.axis_index("c")
    s = lax.axis_index("s")
    wid = s * NC + c
    base = wid * 800

    def body(j, carry):
        r0 = base + j * 4
        pltpu.sync_copy(idx2d.at[pl.ds(r0, 4)], idx_v)
        wa = [pltpu.async_copy(table.at[idx_v.at[r]],
                               rows_a.at[pl.ds(r * 128, 128)], sem_a)
              for r in range(2)]
        wb = [pltpu.async_copy(table.at[idx_v.at[2 + r]],
                               rows_b.at[pl.ds(r * 128, 128)], sem_b)
              for r in range(2)]
        for wdesc in wa:
            wdesc.wait()
        for e in range(256):
            cmp_v[e, pl.ds(0, 16)] = rows_a[e, pl.ds(0, 16)]
            cmp_v[e, pl.ds(16, 16)] = rows_a[e, pl.ds(16, 16)]
        pltpu.sync_copy(cmp_v, out.at[pl.ds(r0 * 128, 256)])
        for wdesc in wb:
            wdesc.wait()
        for e in range(256):
            cmp_v[e, pl.ds(0, 16)] = rows_b[e, pl.ds(0, 16)]
            cmp_v[e, pl.ds(16, 16)] = rows_b[e, pl.ds(16, 16)]
        pltpu.sync_copy(cmp_v, out.at[pl.ds(r0 * 128 + 256, 256)])
        return carry

    lax.fori_loop(0, 200, body, 0)


def _sc_gather(table128, idx2d):
    kern = pl.kernel(
        _sc_gather_body,
        out_type=jax.ShapeDtypeStruct((2 * EP, 32), jnp.float32),
        mesh=_mesh(),
        scratch_types=[
            pltpu.VMEM((4, 128), jnp.int32),
            pltpu.VMEM((256, 128), jnp.float32),
            pltpu.VMEM((256, 128), jnp.float32),
            pltpu.VMEM((256, 32), jnp.float32),
            pltpu.SemaphoreType.DMA,
            pltpu.SemaphoreType.DMA,
        ],
    )
    return kern(table128, idx2d)


def _edge_body(f_ref, gs_ref, gd_ref, w1, b1, w2, b2, w3, b3, w4, b4,
               wm1, bm1, wm2, bm2, lo_ref, hi_ref):
    h = _relu(_dot(f_ref[...], w1[...]) + b1[...])
    h = _relu(_dot(h, w2[...]) + b2[...])
    h = _relu(_dot(h, w3[...]) + b3[...])
    ence = _dot(h, w4[...]) + b4[...]
    cat = jnp.concatenate([gs_ref[:, :20], gd_ref[:, :20], ence], axis=1)
    hm = _relu(_dot(cat, wm1[...]) + bm1[...])
    msg = _dot(hm, wm2[...]) + bm2[...]
    z = jnp.zeros((msg.shape[0], 6), jnp.float32)
    lo_ref[...] = jnp.concatenate([msg[:, :10], z], axis=1)
    hi_ref[...] = jnp.concatenate([msg[:, 10:], z], axis=1)


def _edge_stage(feats, g, we, be_, wm1, bm1, wm2, bm2):
    full = lambda shape: pl.BlockSpec(shape, lambda i: (0,) * len(shape))
    in_specs = [
        pl.BlockSpec((BE, 5), lambda i: (i, 0)),
        pl.BlockSpec((BE, 32), lambda i: (i, 0)),
        pl.BlockSpec((BE, 32), lambda i: (i + OFF_DST, 0)),
    ]
    args = [feats, g, g]
    for w, b in zip(we, be_):
        in_specs += [full(w.shape), full(b.shape)]
        args += [w, b]
    for t in (wm1, bm1, wm2, bm2):
        in_specs.append(full(t.shape))
        args.append(t)
    return pl.pallas_call(
        _edge_body,
        grid=(GE,),
        in_specs=in_specs,
        out_specs=[pl.BlockSpec((BE, 16), lambda i: (i, 0)),
                   pl.BlockSpec((BE, 16), lambda i: (i, 0))],
        out_shape=[jax.ShapeDtypeStruct((EP, 16), jnp.float32),
                   jax.ShapeDtypeStruct((EP, 16), jnp.float32)],
    )(*args)


def _sc_scatter(mlo_pk, mhi_pk, dst2d):
    HR = NP // 2
    DTOP = HR + 128

    def body(mlo_hbm, mhi_hbm, dst2d_hbm, plo_hbm, phi_hbm,
             idx_v, idx2_v, raw_v, msg_v, zv_v, av_v, pk_v, acc, sem):
        c = lax.axis_index("c")
        s = lax.axis_index("s")
        wid = s * NC + c
        stripe = HR // NS
        r0 = s * stripe
        base = wid * 400

        def zfill(i, carry):
            zv_v[i, pl.ds(0, 16)] = jnp.zeros((16,), jnp.float32)
            return carry
        lax.fori_loop(0, 784, zfill, 0)

        for half in (0, 1):
            off = half * HR
            for msg_hbm, part_hbm in ((mlo_hbm, plo_hbm), (mhi_hbm, phi_hbm)):
                def zloop(k, carry):
                    pltpu.sync_copy(zv_v, acc.at[pl.ds(r0 + k * 784, 784)])
                    return carry
                lax.fori_loop(0, 4, zloop, 0)
                pltpu.sync_copy(zv_v.at[pl.ds(0, 8)],
                                acc.at[pl.ds(HR + s * 8, 8)])
                plsc.subcore_barrier()

                def loop(j, carry):
                    rr = base + j * 4
                    pltpu.sync_copy(dst2d_hbm.at[pl.ds(rr, 4)], idx_v)
                    pltpu.sync_copy(msg_hbm.at[pl.ds(rr * 16, 64)], raw_v)
                    for q in range(4):
                        for t in range(8):
                            v = idx_v[q, pl.ds(t * 16, 16)]
                            lo = v - off
                            oob = (lo < 0) | (lo >= HR)
                            dump = HR + (v & 127)
                            idx2_v[q, pl.ds(t * 16, 16)] = jnp.where(oob, dump,
                                                                     lo)
                    for r in range(512):
                        msg_v[r, pl.ds(0, 16)] = raw_v[r // 8,
                                                       pl.ds((r % 8) * 16, 16)]
                    waits = []
                    for q in range(4):
                        waits.append(
                            pltpu.async_copy(msg_v.at[pl.ds(q * 128, 128)],
                                             acc.at[idx2_v.at[q]], sem,
                                             add=True))
                    for wd in waits:
                        wd.wait()
                    return carry

                lax.fori_loop(0, 100, loop, 0)
                plsc.subcore_barrier()

                def wloop(k, carry):
                    pltpu.sync_copy(acc.at[pl.ds(r0 + k * 784, 784)], av_v)
                    for r in range(98):
                        for q in range(8):
                            pk_v[r, pl.ds(q * 16, 16)] = av_v[r * 8 + q,
                                                              pl.ds(0, 16)]
                    pltpu.sync_copy(
                        pk_v,
                        part_hbm.at[c].at[pl.ds(half * 6272 + s * 392 + k * 98,
                                                98)])
                    return carry
                lax.fori_loop(0, 4, wloop, 0)
                plsc.subcore_barrier()

    kern = pl.kernel(
        body,
        out_type=[jax.ShapeDtypeStruct((NC, NP // 8, 128), jnp.float32),
                  jax.ShapeDtypeStruct((NC, NP // 8, 128), jnp.float32)],
        mesh=_mesh(),
        scratch_types=[
            pltpu.VMEM((4, 128), jnp.int32),
            pltpu.VMEM((4, 128), jnp.int32),
            pltpu.VMEM((64, 128), jnp.float32),
            pltpu.VMEM((512, 16), jnp.float32),
            pltpu.VMEM((784, 16), jnp.float32),
            pltpu.VMEM((784, 16), jnp.float32),
            pltpu.VMEM((98, 128), jnp.float32),
            pltpu.VMEM_SHARED((NP // 2 + 128, 16), jnp.float32),
            pltpu.SemaphoreType.DMA,
        ],
        compiler_params=pltpu.CompilerParams(use_tc_tiling_on_sc=False),
    )
    return kern(mlo_pk, mhi_pk, dst2d)


def _node_out_body(enc_ref, lo0_ref, lo1_ref, hi0_ref, hi1_ref, v1, c1, v2, c2,
                   d1, e1, d2, e2, d3, e3, d4, e4, out_ref):
    enc = enc_ref[:, :20]
    agg_lo = (lo0_ref[0] + lo1_ref[0])[:, :10]
    agg_hi = (hi0_ref[0] + hi1_ref[0])[:, :10]
    cat = jnp.concatenate([enc, agg_lo, agg_hi], axis=1)
    h = _relu(_dot(cat, v1[...]) + c1[...])
    pred = enc + _dot(h, v2[...]) + c2[...]
    h = _relu(_dot(pred, d1[...]) + e1[...])
    h = _relu(_dot(h, d2[...]) + e2[...])
    h = _relu(_dot(h, d3[...]) + e3[...])
    out_ref[...] = _dot(h, d4[...]) + e4[...]


def _node_out(table128, parts_lo, parts_hi, v1, c1, v2, c2, wd, bd):
    full = lambda shape: pl.BlockSpec(shape, lambda i: (0,) * len(shape))
    in_specs = [
        pl.BlockSpec((BN, 128), lambda i: (i, 0)),
        pl.BlockSpec((1, BN, 16), lambda i: (0, i, 0)),
        pl.BlockSpec((1, BN, 16), lambda i: (1, i, 0)),
        pl.BlockSpec((1, BN, 16), lambda i: (0, i, 0)),
        pl.BlockSpec((1, BN, 16), lambda i: (1, i, 0)),
    ]
    args = [table128, parts_lo, parts_lo, parts_hi, parts_hi]
    for t in (v1, c1, v2, c2):
        in_specs.append(full(t.shape))
        args.append(t)
    for w, b in zip(wd, bd):
        in_specs += [full(w.shape), full(b.shape)]
        args += [w, b]
    return pl.pallas_call(
        _node_out_body,
        grid=(GN,),
        in_specs=in_specs,
        out_specs=pl.BlockSpec((BN, 3), lambda i: (i, 0)),
        out_shape=jax.ShapeDtypeStruct((N, 3), jnp.float32),
    )(*args)


def kernel(x, features_of_edges, node_indexes_connected_by_edges, batch, params):
    p = params
    r1 = lambda b: b.reshape(1, -1)

    w1m = p["mp_edge_model"]["Ws"][0]
    b1m = p["mp_edge_model"]["bs"][0]
    w2m = p["mp_edge_model"]["Ws"][1]
    b2m = p["mp_edge_model"]["bs"][1]

    we = p["edge_encoder"]["Ws"]
    bbe = p["edge_encoder"]["bs"]

    src = node_indexes_connected_by_edges[0]
    dst = node_indexes_connected_by_edges[1]
    pad_i = jnp.arange(EP - E, dtype=jnp.int32)
    gpad = pad_i % N
    gidx = jnp.concatenate([src, gpad, dst, gpad])
    idx2d = gidx.reshape(25600, 128)
    dst_pad = jnp.concatenate([dst, N + pad_i % (NP - N)])
    dst2d = dst_pad.reshape(12800, 128)

    table128 = _node_encode(x, p["node_encoder"]["Ws"],
                            [r1(b) for b in p["node_encoder"]["bs"]])
    g = _sc_gather(table128, idx2d)
    mlo, mhi = _edge_stage(features_of_edges, g,
                           we, [r1(b) for b in bbe],
                           w1m, r1(b1m), w2m, r1(b2m))
    mlo_pk = mlo.reshape(EP // 8, 128)
    mhi_pk = mhi.reshape(EP // 8, 128)
    plo_pk, phi_pk = _sc_scatter(mlo_pk, mhi_pk, dst2d)
    parts_lo = plo_pk.reshape(NC, NP, 16)
    parts_hi = phi_pk.reshape(NC, NP, 16)

    v1 = p["mp_node_model"]["Ws"][0]
    c1 = p["mp_node_model"]["bs"][0]
    v2 = p["mp_node_model"]["Ws"][1]
    c2 = p["mp_node_model"]["bs"][1]
    out = _node_out(table128, parts_lo, parts_hi, v1, r1(c1), v2, r1(c2),
                    p["node_decoder"]["Ws"],
                    [r1(b) for b in p["node_decoder"]["bs"]])
    return out


# speedup vs baseline: 2.4308x; 1.0507x over previous
import functools

import jax
import jax.numpy as jnp
from jax import lax
from jax.experimental import pallas as pl
from jax.experimental.pallas import tpu as pltpu
from jax.experimental.pallas import tpu_sc as plsc

N = 100000
NP = 100352
E = 1600000
EP = 1638400
NC, NS = 2, 16
BE = 4096
GE = (E + BE - 1) // BE
OFF_DST = EP // BE
BN = 2048
GN = NP // BN


def _mesh():
    return plsc.VectorSubcoreMesh(core_axis_name="c", subcore_axis_name="s",
                                  num_cores=NC, num_subcores=NS)


def _relu(v):
    return jnp.maximum(v, 0.0)


def _dot(a, b):
    return jnp.dot(a, b, preferred_element_type=jnp.float32)


def _node_enc_body(x_ref, w1, b1, w2, b2, w3, b3, w4, b4, out_ref):
    h = _relu(_dot(x_ref[...], w1[...]) + b1[...])
    h = _relu(_dot(h, w2[...]) + b2[...])
    h = _relu(_dot(h, w3[...]) + b3[...])
    enc = _dot(h, w4[...]) + b4[...]
    out_ref[...] = jnp.pad(enc, ((0, 0), (0, 108)))


def _node_encode(x, ws, bs):
    full = lambda shape: pl.BlockSpec(shape, lambda i: (0,) * len(shape))
    in_specs = [pl.BlockSpec((BN, 4), lambda i: (i, 0))]
    args = [x]
    for w, b in zip(ws, bs):
        in_specs += [full(w.shape), full(b.shape)]
        args += [w, b]
    return pl.pallas_call(
        _node_enc_body,
        grid=(GN,),
        in_specs=in_specs,
        out_specs=pl.BlockSpec((BN, 128), lambda i: (i, 0)),
        out_shape=jax.ShapeDtypeStruct((NP, 128), jnp.float32),
    )(*args)


def _sc_gather_body(table, idx2d, out, idx_v, rows_a, rows_b, cmp_v,
                    sem_a, sem_b):
    c = lax.axis_index("c")
    s = lax.axis_index("s")
    wid = s * NC + c
    base = wid * 800

    def body(j, carry):
        r0 = base + j * 4
        pltpu.sync_copy(idx2d.at[pl.ds(r0, 4)], idx_v)
        wa = [pltpu.async_copy(table.at[idx_v.at[r]],
                               rows_a.at[pl.ds(r * 128, 128)], sem_a)
              for r in range(2)]
        wb = [pltpu.async_copy(table.at[idx_v.at[2 + r]],
                               rows_b.at[pl.ds(r * 128, 128)], sem_b)
              for r in range(2)]
        for wdesc in wa:
            wdesc.wait()
        for e in range(256):
            cmp_v[e, pl.ds(0, 16)] = rows_a[e, pl.ds(0, 16)]
            cmp_v[e, pl.ds(16, 16)] = rows_a[e, pl.ds(16, 16)]
        pltpu.sync_copy(cmp_v, out.at[pl.ds(r0 * 128, 256)])
        for wdesc in wb:
            wdesc.wait()
        for e in range(256):
            cmp_v[e, pl.ds(0, 16)] = rows_b[e, pl.ds(0, 16)]
            cmp_v[e, pl.ds(16, 16)] = rows_b[e, pl.ds(16, 16)]
        pltpu.sync_copy(cmp_v, out.at[pl.ds(r0 * 128 + 256, 256)])
        return carry

    lax.fori_loop(0, 200, body, 0)


def _sc_gather(table128, idx2d):
    kern = pl.kernel(
        _sc_gather_body,
        out_type=jax.ShapeDtypeStruct((2 * EP, 32), jnp.float32),
        mesh=_mesh(),
        scratch_types=[
            pltpu.VMEM((4, 128), jnp.int32),
            pltpu.VMEM((256, 128), jnp.float32),
            pltpu.VMEM((256, 128), jnp.float32),
            pltpu.VMEM((256, 32), jnp.float32),
            pltpu.SemaphoreType.DMA,
            pltpu.SemaphoreType.DMA,
        ],
    )
    return kern(table128, idx2d)


def _edge_body(f_ref, gs_ref, gd_ref, w1, b1, w2, b2, w3, b3, w4, b4,
               wm1, bm1, wm2, bm2, lo_ref, hi_ref):
    h = _relu(_dot(f_ref[...], w1[...]) + b1[...])
    h = _relu(_dot(h, w2[...]) + b2[...])
    h = _relu(_dot(h, w3[...]) + b3[...])
    ence = _dot(h, w4[...]) + b4[...]
    cat = jnp.concatenate([gs_ref[:, :20], gd_ref[:, :20], ence], axis=1)
    hm = _relu(_dot(cat, wm1[...]) + bm1[...])
    msg = _dot(hm, wm2[...]) + bm2[...]
    z = jnp.zeros((msg.shape[0], 6), jnp.float32)
    lo_ref[...] = jnp.concatenate([msg[:, :10], z], axis=1)
    hi_ref[...] = jnp.concatenate([msg[:, 10:], z], axis=1)


def _edge_stage(feats, g, we, be_, wm1, bm1, wm2, bm2):
    full = lambda shape: pl.BlockSpec(shape, lambda i: (0,) * len(shape))
    in_specs = [
        pl.BlockSpec((BE, 5), lambda i: (i, 0)),
        pl.BlockSpec((BE, 32), lambda i: (i, 0)),
        pl.BlockSpec((BE, 32), lambda i: (i + OFF_DST, 0)),
    ]
    args = [feats, g, g]
    for w, b in zip(we, be_):
        in_specs += [full(w.shape), full(b.shape)]
        args += [w, b]
    for t in (wm1, bm1, wm2, bm2):
        in_specs.append(full(t.shape))
        args.append(t)
    return pl.pallas_call(
        _edge_body,
        grid=(GE,),
        in_specs=in_specs,
        out_specs=[pl.BlockSpec((BE, 16), lambda i: (i, 0)),
                   pl.BlockSpec((BE, 16), lambda i: (i, 0))],
        out_shape=[jax.ShapeDtypeStruct((EP, 16), jnp.float32),
                   jax.ShapeDtypeStruct((EP, 16), jnp.float32)],
    )(*args)


def _sc_scatter(mlo_pk, mhi_pk, dst2d):
    HR = NP // 2
    DTOP = HR + 128

    def body(mlo_hbm, mhi_hbm, dst2d_hbm, plo_hbm, phi_hbm,
             idx_v, idx2_v, raw_v, msg_v, zv_v, av_v, pk_v, acc, sem):
        c = lax.axis_index("c")
        s = lax.axis_index("s")
        wid = s * NC + c
        stripe = HR // NS
        r0 = s * stripe
        base = wid * 400

        def zfill(i, carry):
            zv_v[i, pl.ds(0, 16)] = jnp.zeros((16,), jnp.float32)
            return carry
        lax.fori_loop(0, 784, zfill, 0)

        for half in (0, 1):
            off = half * HR
            for msg_hbm, part_hbm in ((mlo_hbm, plo_hbm), (mhi_hbm, phi_hbm)):
                def zloop(k, carry):
                    pltpu.sync_copy(zv_v, acc.at[pl.ds(r0 + k * 784, 784)])
                    return carry
                lax.fori_loop(0, 4, zloop, 0)
                pltpu.sync_copy(zv_v.at[pl.ds(0, 8)],
                                acc.at[pl.ds(HR + s * 8, 8)])
                plsc.subcore_barrier()

                def loop(j, carry):
                    rr = base + j * 4
                    pltpu.sync_copy(dst2d_hbm.at[pl.ds(rr, 4)], idx_v)
                    pltpu.sync_copy(msg_hbm.at[pl.ds(rr * 16, 64)], raw_v)
                    for q in range(4):
                        for t in range(8):
                            v = idx_v[q, pl.ds(t * 16, 16)]
                            lo = v - off
                            oob = (lo < 0) | (lo >= HR)
                            dump = HR + (v & 127)
                            idx2_v[q, pl.ds(t * 16, 16)] = jnp.where(oob, dump,
                                                                     lo)
                    for r in range(512):
                        msg_v[r, pl.ds(0, 16)] = raw_v[r // 8,
                                                       pl.ds((r % 8) * 16, 16)]
                    waits = []
                    for q in range(4):
                        waits.append(
                            pltpu.async_copy(msg_v.at[pl.ds(q * 128, 128)],
                                             acc.at[idx2_v.at[q]], sem,
                                             add=True))
                    for wd in waits:
                        wd.wait()
                    return carry

                lax.fori_loop(0, 100, loop, 0)
                plsc.subcore_barrier()

                def wloop(k, carry):
                    pltpu.sync_copy(acc.at[pl.ds(r0 + k * 784, 784)], av_v)
                    for r in range(98):
                        for q in range(8):
                            pk_v[r, pl.ds(q * 16, 16)] = av_v[r * 8 + q,
                                                              pl.ds(0, 16)]
                    pltpu.sync_copy(
                        pk_v,
                        part_hbm.at[c].at[pl.ds(half * 6272 + s * 392 + k * 98,
                                                98)])
                    return carry
                lax.fori_loop(0, 4, wloop, 0)
                plsc.subcore_barrier()

    kern = pl.kernel(
        body,
        out_type=[jax.ShapeDtypeStruct((NC, NP // 8, 128), jnp.float32),
                  jax.ShapeDtypeStruct((NC, NP // 8, 128), jnp.float32)],
        mesh=_mesh(),
        scratch_types=[
            pltpu.VMEM((4, 128), jnp.int32),
            pltpu.VMEM((4, 128), jnp.int32),
            pltpu.VMEM((64, 128), jnp.float32),
            pltpu.VMEM((512, 16), jnp.float32),
            pltpu.VMEM((784, 16), jnp.float32),
            pltpu.VMEM((784, 16), jnp.float32),
            pltpu.VMEM((98, 128), jnp.float32),
            pltpu.VMEM_SHARED((NP // 2 + 128, 16), jnp.float32),
            pltpu.SemaphoreType.DMA,
        ],
        compiler_params=pltpu.CompilerParams(use_tc_tiling_on_sc=False),
    )
    return kern(mlo_pk, mhi_pk, dst2d)


def _node_out_body(enc_ref, lo0_ref, lo1_ref, hi0_ref, hi1_ref, v1, c1, v2, c2,
                   d1, e1, d2, e2, d3, e3, d4, e4, out_ref):
    enc = enc_ref[:, :20]
    agg_lo = (lo0_ref[0] + lo1_ref[0])[:, :10]
    agg_hi = (hi0_ref[0] + hi1_ref[0])[:, :10]
    cat = jnp.concatenate([enc, agg_lo, agg_hi], axis=1)
    h = _relu(_dot(cat, v1[...]) + c1[...])
    pred = enc + _dot(h, v2[...]) + c2[...]
    h = _relu(_dot(pred, d1[...]) + e1[...])
    h = _relu(_dot(h, d2[...]) + e2[...])
    h = _relu(_dot(h, d3[...]) + e3[...])
    out_ref[...] = _dot(h, d4[...]) + e4[...]


def _node_out(table128, parts_lo, parts_hi, v1, c1, v2, c2, wd, bd):
    full = lambda shape: pl.BlockSpec(shape, lambda i: (0,) * len(shape))
    in_specs = [
        pl.BlockSpec((BN, 128), lambda i: (i, 0)),
        pl.BlockSpec((1, BN, 16), lambda i: (0, i, 0)),
        pl.BlockSpec((1, BN, 16), lambda i: (1, i, 0)),
        pl.BlockSpec((1, BN, 16), lambda i: (0, i, 0)),
        pl.BlockSpec((1, BN, 16), lambda i: (1, i, 0)),
    ]
    args = [table128, parts_lo, parts_lo, parts_hi, parts_hi]
    for t in (v1, c1, v2, c2):
        in_specs.append(full(t.shape))
        args.append(t)
    for w, b in zip(wd, bd):
        in_specs += [full(w.shape), full(b.shape)]
        args += [w, b]
    return pl.pallas_call(
        _node_out_body,
        grid=(GN,),
        in_specs=in_specs,
        out_specs=pl.BlockSpec((BN, 3), lambda i: (i, 0)),
        out_shape=jax.ShapeDtypeStruct((N, 3), jnp.float32),
    )(*args)


def kernel(x, features_of_edges, node_indexes_connected_by_edges, batch, params):
    p = params
    r1 = lambda b: b.reshape(1, -1)

    w1m = p["mp_edge_model"]["Ws"][0]
    b1m = p["mp_edge_model"]["bs"][0]
    w2m = p["mp_edge_model"]["Ws"][1]
    b2m = p["mp_edge_model"]["bs"][1]

    we = p["edge_encoder"]["Ws"]
    bbe = p["edge_encoder"]["bs"]

    src = node_indexes_connected_by_edges[0]
    dst = node_indexes_connected_by_edges[1]
    pad_i = jnp.arange(EP - E, dtype=jnp.int32)
    gpad = pad_i % N
    gidx = jnp.concatenate([src, gpad, dst, gpad])
    idx2d = gidx.reshape(25600, 128)
    dst_pad = jnp.concatenate([dst, N + pad_i % (NP - N)])
    dst2d = dst_pad.reshape(12800, 128)

    table128 = _node_encode(x, p["node_encoder"]["Ws"],
                            [r1(b) for b in p["node_encoder"]["bs"]])
    g = _sc_gather(table128, idx2d)
    mlo, mhi = _edge_stage(features_of_edges, g,
                           we, [r1(b) for b in bbe],
                           w1m, r1(b1m), w2m, r1(b2m))
    mlo_pk = mlo.reshape(EP // 8, 128)
    mhi_pk = mhi.reshape(EP // 8, 128)
    plo_pk, phi_pk = _sc_scatter(mlo_pk, mhi_pk, dst2d)
    parts_lo = plo_pk.reshape(NC, NP, 16)
    parts_hi = phi_pk.reshape(NC, NP, 16)

    v1 = p["mp_node_model"]["Ws"][0]
    c1 = p["mp_node_model"]["bs"][0]
    v2 = p["mp_node_model"]["Ws"][1]
    c2 = p["mp_node_model"]["bs"][1]
    out = _node_out(table128, parts_lo, parts_hi, v1, r1(c1), v2, r1(c2),
                    p["node_decoder"]["Ws"],
                    [r1(b) for b in p["node_decoder"]["bs"]])
    return out
